# Initial kernel scaffold; baseline (speedup 1.0000x reference)
import jax
import jax.numpy as jnp
from jax.experimental import pallas as pl


def kernel(msa_row, pair_rep, ca_coords, W_oh, b_oh, g_pair, bt_pair, g_msa, bt_msa):
    raise NotImplementedError("write your pallas kernel here")


# speedup vs baseline: 1.8747x; 1.8747x over previous
import functools

import jax
import jax.numpy as jnp
import numpy as np
from jax.experimental import pallas as pl

N_RES = 512
PAIR_EMB = 128
MSA_EMB = 256
NBINS = 15
_BINS = np.concatenate(
    [np.array([3.375], dtype=np.float32),
     np.arange(5.125, 22.0, 1.25, dtype=np.float32)]
).astype(np.float32)
assert _BINS.shape[0] == NBINS


def _pair_body(R, coords_ref, crow_ref, w_ref, b_ref, g_ref, bt_ref, pair_ref, out_ref):
    call = coords_ref[:, :]
    crow = crow_ref[:, :]
    d2 = ((crow[:, 0:1] - call[0:1, :]) ** 2
          + (crow[:, 1:2] - call[1:2, :]) ** 2
          + (crow[:, 2:3] - call[2:3, :]) ** 2)
    dist = jnp.sqrt(d2)

    best = jnp.abs(dist - _BINS[0])
    idx = jnp.zeros(dist.shape, dtype=jnp.int32)
    for k in range(1, NBINS):
        cand = jnp.abs(dist - _BINS[k])
        take = cand < best
        idx = jnp.where(take, k, idx)
        best = jnp.where(take, cand, best)

    oh = (idx[:, :, None] ==
          jax.lax.broadcasted_iota(jnp.int32, (R, N_RES, NBINS), 2)
          ).astype(jnp.float32)
    emb = jax.lax.dot_general(
        oh.reshape(R * N_RES, NBINS), w_ref[:, :],
        (((1,), (1,)), ((), ())),
        preferred_element_type=jnp.float32,
    ).reshape(R, N_RES, PAIR_EMB) + b_ref[0, :]

    x = pair_ref[...]
    mu = jnp.mean(x, axis=-1, keepdims=True)
    var = jnp.mean((x - mu) ** 2, axis=-1, keepdims=True)
    ln = (x - mu) / jnp.sqrt(var + 1e-5) * g_ref[0, :] + bt_ref[0, :]
    out_ref[...] = ln + emb


def _msa_body(x_ref, g_ref, b_ref, o_ref):
    x = x_ref[...]
    mu = jnp.mean(x, axis=-1, keepdims=True)
    var = jnp.mean((x - mu) ** 2, axis=-1, keepdims=True)
    o_ref[...] = (x - mu) / jnp.sqrt(var + 1e-5) * g_ref[0, :] + b_ref[0, :]


def kernel(msa_row, pair_rep, ca_coords, W_oh, b_oh, g_pair, bt_pair, g_msa, bt_msa):
    R = 16
    coords_t = ca_coords.T
    pair_out = pl.pallas_call(
        functools.partial(_pair_body, R),
        grid=(N_RES // R,),
        in_specs=[
            pl.BlockSpec((3, N_RES), lambda i: (0, 0)),
            pl.BlockSpec((R, 3), lambda i: (i, 0)),
            pl.BlockSpec((PAIR_EMB, NBINS), lambda i: (0, 0)),
            pl.BlockSpec((1, PAIR_EMB), lambda i: (0, 0)),
            pl.BlockSpec((1, PAIR_EMB), lambda i: (0, 0)),
            pl.BlockSpec((1, PAIR_EMB), lambda i: (0, 0)),
            pl.BlockSpec((R, N_RES, PAIR_EMB), lambda i: (i, 0, 0)),
        ],
        out_specs=pl.BlockSpec((R, N_RES, PAIR_EMB), lambda i: (i, 0, 0)),
        out_shape=jax.ShapeDtypeStruct((N_RES, N_RES, PAIR_EMB), jnp.float32),
    )(coords_t, ca_coords, W_oh, b_oh.reshape(1, PAIR_EMB), g_pair.reshape(1, PAIR_EMB),
      bt_pair.reshape(1, PAIR_EMB), pair_rep)

    msa_out = pl.pallas_call(
        _msa_body,
        in_specs=[
            pl.BlockSpec((N_RES, MSA_EMB), lambda: (0, 0)),
            pl.BlockSpec((1, MSA_EMB), lambda: (0, 0)),
            pl.BlockSpec((1, MSA_EMB), lambda: (0, 0)),
        ],
        out_specs=pl.BlockSpec((N_RES, MSA_EMB), lambda: (0, 0)),
        out_shape=jax.ShapeDtypeStruct((N_RES, MSA_EMB), jnp.float32),
    )(msa_row, g_msa.reshape(1, MSA_EMB), bt_msa.reshape(1, MSA_EMB))

    return (msa_out, pair_out)


# speedup vs baseline: 2.7408x; 1.4620x over previous
import functools

import jax
import jax.numpy as jnp
import numpy as np
from jax.experimental import pallas as pl

N_RES = 512
PAIR_EMB = 128
MSA_EMB = 256
NBINS = 15
_BINS = np.concatenate(
    [np.array([3.375], dtype=np.float32),
     np.arange(5.125, 22.0, 1.25, dtype=np.float32)]
).astype(np.float32)
assert _BINS.shape[0] == NBINS


def _pair_body(R, coords_ref, crow_ref, wt_ref, b_ref, g_ref, bt_ref,
               msa_ref, gm_ref, bm_ref, pair_ref, out_ref, msa_out_ref):
    call = coords_ref[:, :]
    crow = crow_ref[:, :]
    d2 = ((crow[:, 0:1] - call[0:1, :]) ** 2
          + (crow[:, 1:2] - call[1:2, :]) ** 2
          + (crow[:, 2:3] - call[2:3, :]) ** 2)
    dist = jnp.sqrt(d2)

    best = jnp.abs(dist - _BINS[0])
    idx = jnp.zeros(dist.shape, dtype=jnp.int32)
    for k in range(1, NBINS):
        cand = jnp.abs(dist - _BINS[k])
        take = cand < best
        idx = jnp.where(take, k, idx)
        best = jnp.where(take, cand, best)

    table = (wt_ref[:, :] + b_ref[0, :] + bt_ref[0, :])
    oh = (idx[:, :, None] ==
          jax.lax.broadcasted_iota(jnp.int32, (R, N_RES, NBINS), 2)
          ).astype(jnp.bfloat16)
    emb = jax.lax.dot_general(
        oh.reshape(R * N_RES, NBINS), table.astype(jnp.bfloat16),
        (((1,), (0,)), ((), ())),
        preferred_element_type=jnp.float32,
    )

    ones_j = jnp.full((PAIR_EMB, PAIR_EMB), 1.0 / PAIR_EMB, dtype=jnp.bfloat16)
    x = pair_ref[...].reshape(R * N_RES, PAIR_EMB)
    xh = x.astype(jnp.bfloat16)
    dims = (((1,), (0,)), ((), ()))
    mu = jax.lax.dot_general(xh, ones_j, dims,
                             preferred_element_type=jnp.float32)
    ex2 = jax.lax.dot_general(xh * xh, ones_j, dims,
                              preferred_element_type=jnp.float32)
    rg = jax.lax.rsqrt(ex2 - mu * mu + 1e-5) * g_ref[0, :]
    out_ref[...] = ((x - mu) * rg + emb).reshape(R, N_RES, PAIR_EMB)

    @pl.when(pl.program_id(0) == 0)
    def _msa():
        ones_m = jnp.full((MSA_EMB, MSA_EMB), 1.0 / MSA_EMB, dtype=jnp.bfloat16)
        m = msa_ref[...]
        mh = m.astype(jnp.bfloat16)
        mdims = (((1,), (0,)), ((), ()))
        mmu = jax.lax.dot_general(mh, ones_m, mdims,
                                  preferred_element_type=jnp.float32)
        mex2 = jax.lax.dot_general(mh * mh, ones_m, mdims,
                                   preferred_element_type=jnp.float32)
        mr = jax.lax.rsqrt(mex2 - mmu * mmu + 1e-5)
        msa_out_ref[...] = (m - mmu) * mr * gm_ref[0, :] + bm_ref[0, :]


def kernel(msa_row, pair_rep, ca_coords, W_oh, b_oh, g_pair, bt_pair, g_msa, bt_msa):
    R = 16
    coords_t = ca_coords.T
    pair_out, msa_out = pl.pallas_call(
        functools.partial(_pair_body, R),
        grid=(N_RES // R,),
        in_specs=[
            pl.BlockSpec((3, N_RES), lambda i: (0, 0)),
            pl.BlockSpec((R, 3), lambda i: (i, 0)),
            pl.BlockSpec((NBINS, PAIR_EMB), lambda i: (0, 0)),
            pl.BlockSpec((1, PAIR_EMB), lambda i: (0, 0)),
            pl.BlockSpec((1, PAIR_EMB), lambda i: (0, 0)),
            pl.BlockSpec((1, PAIR_EMB), lambda i: (0, 0)),
            pl.BlockSpec((N_RES, MSA_EMB), lambda i: (0, 0)),
            pl.BlockSpec((1, MSA_EMB), lambda i: (0, 0)),
            pl.BlockSpec((1, MSA_EMB), lambda i: (0, 0)),
            pl.BlockSpec((R, N_RES, PAIR_EMB), lambda i: (i, 0, 0)),
        ],
        out_specs=[
            pl.BlockSpec((R, N_RES, PAIR_EMB), lambda i: (i, 0, 0)),
            pl.BlockSpec((N_RES, MSA_EMB), lambda i: (0, 0)),
        ],
        out_shape=[
            jax.ShapeDtypeStruct((N_RES, N_RES, PAIR_EMB), jnp.float32),
            jax.ShapeDtypeStruct((N_RES, MSA_EMB), jnp.float32),
        ],
    )(coords_t, ca_coords, W_oh.T, b_oh.reshape(1, PAIR_EMB),
      g_pair.reshape(1, PAIR_EMB), bt_pair.reshape(1, PAIR_EMB),
      msa_row, g_msa.reshape(1, MSA_EMB), bt_msa.reshape(1, MSA_EMB),
      pair_rep)
    return (msa_out, pair_out)


# speedup vs baseline: 2.9963x; 1.0932x over previous
import functools

import jax
import jax.numpy as jnp
import numpy as np
from jax.experimental import pallas as pl

N_RES = 512
PAIR_EMB = 128
MSA_EMB = 256
NBINS = 15
_BINS = np.concatenate(
    [np.array([3.375], dtype=np.float32),
     np.arange(5.125, 22.0, 1.25, dtype=np.float32)]
).astype(np.float32)
assert _BINS.shape[0] == NBINS


def _pair_body(R, coords_ref, crow_ref, wt_ref, b_ref, g_ref, bt_ref,
               msa_ref, gm_ref, bm_ref, pair_ref, out_ref, msa_out_ref):
    call = coords_ref[:, :]
    crow = crow_ref[:, :]
    d2 = ((crow[:, 0:1] - call[0:1, :]) ** 2
          + (crow[:, 1:2] - call[1:2, :]) ** 2
          + (crow[:, 2:3] - call[2:3, :]) ** 2)
    dist = jnp.sqrt(d2)

    best = jnp.abs(dist - _BINS[0])
    idx = jnp.zeros(dist.shape, dtype=jnp.int32)
    for k in range(1, NBINS):
        cand = jnp.abs(dist - _BINS[k])
        take = cand < best
        idx = jnp.where(take, k, idx)
        best = jnp.where(take, cand, best)

    table = (wt_ref[:, :] + b_ref[0, :] + bt_ref[0, :])
    oh = (idx[:, :, None] ==
          jax.lax.broadcasted_iota(jnp.int32, (R, N_RES, NBINS), 2)
          ).astype(jnp.bfloat16)
    emb = jax.lax.dot_general(
        oh.reshape(R * N_RES, NBINS), table.astype(jnp.bfloat16),
        (((1,), (0,)), ((), ())),
        preferred_element_type=jnp.float32,
    )

    ones_j = jnp.full((PAIR_EMB, PAIR_EMB), 1.0 / PAIR_EMB, dtype=jnp.bfloat16)
    x = pair_ref[...].reshape(R * N_RES, PAIR_EMB)
    xh = x.astype(jnp.bfloat16)
    dims = (((1,), (0,)), ((), ()))
    mu = jax.lax.dot_general(xh, ones_j, dims,
                             preferred_element_type=jnp.float32)
    ex2 = jax.lax.dot_general(xh * xh, ones_j, dims,
                              preferred_element_type=jnp.float32)
    rg = jax.lax.rsqrt(ex2 - mu * mu + 1e-5) * g_ref[0, :]
    out_ref[...] = ((x - mu) * rg + emb).reshape(R, N_RES, PAIR_EMB)

    @pl.when(pl.program_id(0) == 0)
    def _msa():
        ones_m = jnp.full((MSA_EMB, MSA_EMB), 1.0 / MSA_EMB, dtype=jnp.bfloat16)
        m = msa_ref[...]
        mh = m.astype(jnp.bfloat16)
        mdims = (((1,), (0,)), ((), ()))
        mmu = jax.lax.dot_general(mh, ones_m, mdims,
                                  preferred_element_type=jnp.float32)
        mex2 = jax.lax.dot_general(mh * mh, ones_m, mdims,
                                   preferred_element_type=jnp.float32)
        mr = jax.lax.rsqrt(mex2 - mmu * mmu + 1e-5)
        msa_out_ref[...] = (m - mmu) * mr * gm_ref[0, :] + bm_ref[0, :]


def kernel(msa_row, pair_rep, ca_coords, W_oh, b_oh, g_pair, bt_pair, g_msa, bt_msa):
    R = 32
    coords_t = ca_coords.T
    pair_out, msa_out = pl.pallas_call(
        functools.partial(_pair_body, R),
        grid=(N_RES // R,),
        in_specs=[
            pl.BlockSpec((3, N_RES), lambda i: (0, 0)),
            pl.BlockSpec((R, 3), lambda i: (i, 0)),
            pl.BlockSpec((NBINS, PAIR_EMB), lambda i: (0, 0)),
            pl.BlockSpec((1, PAIR_EMB), lambda i: (0, 0)),
            pl.BlockSpec((1, PAIR_EMB), lambda i: (0, 0)),
            pl.BlockSpec((1, PAIR_EMB), lambda i: (0, 0)),
            pl.BlockSpec((N_RES, MSA_EMB), lambda i: (0, 0)),
            pl.BlockSpec((1, MSA_EMB), lambda i: (0, 0)),
            pl.BlockSpec((1, MSA_EMB), lambda i: (0, 0)),
            pl.BlockSpec((R, N_RES, PAIR_EMB), lambda i: (i, 0, 0)),
        ],
        out_specs=[
            pl.BlockSpec((R, N_RES, PAIR_EMB), lambda i: (i, 0, 0)),
            pl.BlockSpec((N_RES, MSA_EMB), lambda i: (0, 0)),
        ],
        out_shape=[
            jax.ShapeDtypeStruct((N_RES, N_RES, PAIR_EMB), jnp.float32),
            jax.ShapeDtypeStruct((N_RES, MSA_EMB), jnp.float32),
        ],
    )(coords_t, ca_coords, W_oh.T, b_oh.reshape(1, PAIR_EMB),
      g_pair.reshape(1, PAIR_EMB), bt_pair.reshape(1, PAIR_EMB),
      msa_row, g_msa.reshape(1, MSA_EMB), bt_msa.reshape(1, MSA_EMB),
      pair_rep)
    return (msa_out, pair_out)
